# baseline (speedup 1.0000x reference)
import jax
import jax.numpy as jnp
from jax import lax
from jax.experimental import pallas as pl
from jax.experimental.pallas import tpu as pltpu

BS, C, H, W = 64, 3, 512, 512

PR = 64
SH = H + 2 * PR
HH = H // 2


def _diffaug_kernel(br_ref, sr_ref, cr_ref, dh_ref, dw_ref, x_ref, o_ref,
                    s_ref):
    i = pl.program_id(0)

    @pl.when(i == 0)
    def _zero_pads():
        s_ref[:, 0:PR, :] = jnp.zeros((C, PR, W), jnp.float32)
        s_ref[:, PR + H:SH, :] = jnp.zeros((C, PR, W), jnp.float32)

    br = br_ref[i]
    sr = sr_ref[i]
    cr = cr_ref[i]
    dh = dh_ref[i]
    dw = dw_ref[i]

    xb = x_ref[0]
    mc = (xb[0] + xb[1] + xb[2]) * (1.0 / 3.0)
    M = jnp.mean(mc)

    cs = cr + 0.5
    A = cs * 2.0 * sr
    B = cs * (1.0 - 2.0 * sr)
    Cc = M * (0.5 - cr) + br - 0.5
    t = B * mc + Cc

    cols = lax.broadcasted_iota(jnp.int32, (H, W), 1)
    cvalid = (cols + dw >= 0) & (cols + dw < W)

    start = PR + dh
    rr = lax.rem(start, 8)
    base = pl.multiple_of(start - rr, 8)

    def _affine_part(lo, n):
        o3 = A * xb[:, lo:lo + n, :] + t[None, lo:lo + n, :]
        o3 = jnp.where(cvalid[None, lo:lo + n, :],
                       pltpu.roll(o3, -dw, 2), 0.0)
        s_ref[:, PR + lo:PR + lo + n, :] = o3

    def _copy_half(lo):
        for r in range(8):
            @pl.when(rr == r)
            def _copy(r=r):
                v = s_ref[:, pl.ds(base + lo, HH + 8), :]
                o_ref[0, :, lo:lo + HH, :] = v[:, r:r + HH, :]

    _affine_part(0, 320)
    _copy_half(0)
    _affine_part(320, 192)
    _copy_half(HH)


@jax.jit
def kernel(x, b_rand, s_rand, c_rand, dh, dw):
    br = b_rand.reshape(BS).astype(jnp.float32)
    sr = s_rand.reshape(BS).astype(jnp.float32)
    cr = c_rand.reshape(BS).astype(jnp.float32)
    dhi = dh.reshape(BS).astype(jnp.int32)
    dwi = dw.reshape(BS).astype(jnp.int32)

    grid_spec = pltpu.PrefetchScalarGridSpec(
        num_scalar_prefetch=5,
        grid=(BS,),
        in_specs=[
            pl.BlockSpec((1, C, H, W), lambda i, *_: (i, 0, 0, 0)),
        ],
        out_specs=pl.BlockSpec((1, C, H, W), lambda i, *_: (i, 0, 0, 0)),
        scratch_shapes=[pltpu.VMEM((C, SH, W), jnp.float32)],
    )
    return pl.pallas_call(
        _diffaug_kernel,
        grid_spec=grid_spec,
        out_shape=jax.ShapeDtypeStruct((BS, C, H, W), jnp.float32),
    )(br, sr, cr, dhi, dwi, x)
